# Initial kernel scaffold; baseline (speedup 1.0000x reference)
import jax
import jax.numpy as jnp
from jax.experimental import pallas as pl


def kernel(x, edge_index, W1, as1, ad1, b1, W2, as2, ad2, b2, W3, b3):
    raise NotImplementedError("write your pallas kernel here")


# speedup vs baseline: 4.3039x; 4.3039x over previous
import functools

import jax
import jax.numpy as jnp
from jax import lax
from jax.experimental import pallas as pl
from jax.experimental.pallas import tpu as pltpu
from jax.experimental.pallas import tpu_sc as plsc

HEADS = 8
C = 128
F = HEADS * C
DW = F + 128
CH = 80
NW = 32
B = 8


def _mm_att(xin, W, Amat, bn):
    n, k = xin.shape

    def body(x_ref, w_ref, a_ref, t_ref, d_ref):
        h = jnp.dot(x_ref[...], w_ref[...], preferred_element_type=jnp.float32)
        aa = jnp.dot(h, a_ref[...], preferred_element_type=jnp.float32)
        t_ref[:, :F] = h
        t_ref[:, F:F + 16] = aa[:, 0:16]
        d_ref[...] = aa[:, 16:32]

    return pl.pallas_call(
        body,
        grid=(n // bn,),
        in_specs=[
            pl.BlockSpec((bn, k), lambda i: (i, 0)),
            pl.BlockSpec((k, F), lambda i: (0, 0)),
            pl.BlockSpec((F, 32), lambda i: (0, 0)),
        ],
        out_specs=[
            pl.BlockSpec((bn, DW), lambda i: (i, 0)),
            pl.BlockSpec((bn, 16), lambda i: (i, 0)),
        ],
        out_shape=[
            jax.ShapeDtypeStruct((n, DW), jnp.float32),
            jax.ShapeDtypeStruct((n, 16), jnp.float32),
        ],
    )(xin, W, Amat)


def _final_mm(g, W3p, b3p, bn):
    n, k = g.shape

    def body(x_ref, w_ref, b_ref, o_ref):
        o_ref[...] = (
            jnp.dot(x_ref[...], w_ref[...], preferred_element_type=jnp.float32)
            + b_ref[...]
        )

    return pl.pallas_call(
        body,
        grid=(n // bn,),
        in_specs=[
            pl.BlockSpec((bn, k), lambda i: (i, 0)),
            pl.BlockSpec((k, 128), lambda i: (0, 0)),
            pl.BlockSpec((1, 128), lambda i: (0, 0)),
        ],
        out_specs=pl.BlockSpec((bn, 128), lambda i: (i, 0)),
        out_shape=jax.ShapeDtypeStruct((n, 128), jnp.float32),
    )(g, W3p, b3p)


def _gat_sc(T, srcP, dstP, adflat, bnd, bias, n):
    nc = n // CH
    ncw = (nc + NW - 1) // NW

    mesh = plsc.VectorSubcoreMesh(
        core_axis_name="c", subcore_axis_name="s", num_cores=2, num_subcores=16)

    @functools.partial(
        pl.kernel,
        out_type=jax.ShapeDtypeStruct((n * F,), jnp.float32),
        mesh=mesh,
        scratch_types=[
            pltpu.VMEM((CH * F,), jnp.float32),
            pltpu.VMEM((CH * 16,), jnp.float32),
            pltpu.VMEM((CH * 16,), jnp.float32),
            pltpu.VMEM((B, DW), jnp.float32),
            pltpu.VMEM((B,), jnp.int32),
            pltpu.VMEM((16,), jnp.int32),
            pltpu.VMEM((160,), jnp.int32),
            pltpu.VMEM((F,), jnp.float32),
            pltpu.SemaphoreType.DMA,
        ],
    )
    def sc_kernel(t_h, src_h, dst_h, ad_h, bnd_h, b_h, g_h,
                  table, denomt, adstv, rows, idxv, dstv, bndv, biasv,
                  sem):
        cid = lax.axis_index("c")
        sid = lax.axis_index("s")
        wid = sid * 2 + cid
        pltpu.sync_copy(bnd_h, bndv)
        pltpu.sync_copy(b_h, biasv)

        @pl.loop(0, ncw)
        def _chunk_iter(ci):
            chunk = wid + ci * NW

            @pl.when(chunk < nc)
            def _():
                base = chunk * CH
                e01 = bndv[pl.ds(chunk, 16)]
                e0 = e01[0]
                e1 = e01[1]
                pltpu.sync_copy(ad_h.at[pl.ds(base * 16, CH * 16)], adstv)

                @pl.loop(0, CH * (F // 16))
                def _zero_table(i):
                    table[pl.ds(i * 16, 16)] = jnp.zeros((16,), jnp.float32)

                @pl.loop(0, CH)
                def _zero_denom(d):
                    denomt[pl.ds(d * 16, 16)] = jnp.zeros((16,), jnp.float32)

                e0a = (e0 // B) * B
                nb = (e1 - e0a + (B - 1)) // B

                @pl.loop(0, nb)
                def _batch(bi):
                    k = e0a + bi * B
                    pltpu.sync_copy(src_h.at[pl.ds(k, B)], idxv)
                    pltpu.sync_copy(dst_h.at[pl.ds(k, B)], dstv.at[pl.ds(0, B)])
                    pltpu.async_copy(t_h.at[idxv], rows, sem).wait()
                    dv = dstv[...]
                    for j in range(B):
                        local = dv[j] - base

                        @pl.when(jnp.logical_and(local >= 0, local < CH))
                        def _():
                            tail = rows[j, pl.ds(F, 16)]
                            adr = adstv[pl.ds(local * 16, 16)]
                            alpha = tail + adr
                            alpha = jnp.where(alpha >= 0.0, alpha, 0.2 * alpha)
                            ex = jnp.exp(alpha)
                            doff = local * 16
                            denomt[pl.ds(doff, 16)] = (
                                denomt[pl.ds(doff, 16)] + ex)
                            toff = local * F
                            for h in range(HEADS):
                                bc = jnp.full((16,), ex[h], jnp.float32)
                                for c in range(C // 16):
                                    off = h * C + c * 16
                                    table[pl.ds(toff + off, 16)] = (
                                        table[pl.ds(toff + off, 16)]
                                        + bc * rows[j, pl.ds(off, 16)])

                @pl.loop(0, CH)
                def _finalize(d):
                    dn = denomt[pl.ds(d * 16, 16)]
                    rec = 1.0 / (dn + 1e-16)
                    for h in range(HEADS):
                        bc = jnp.full((16,), rec[h], jnp.float32)
                        for c in range(C // 16):
                            off = h * C + c * 16
                            v = (table[pl.ds(d * F + off, 16)] * bc
                                 + biasv[pl.ds(off, 16)])
                            v = jnp.where(v > 0.0, v, jnp.exp(v) - 1.0)
                            table[pl.ds(d * F + off, 16)] = v

                pltpu.sync_copy(table, g_h.at[pl.ds(base * F, CH * F)])

    return sc_kernel(T, srcP, dstP, adflat, bnd, bias)


def _att_mat(a_s, a_d):
    eye = jnp.eye(HEADS, dtype=jnp.float32)
    As = jnp.einsum("hc,hk->hck", a_s, eye).reshape(F, HEADS)
    Ad = jnp.einsum("hc,hk->hck", a_d, eye).reshape(F, HEADS)
    z8 = jnp.zeros((F, HEADS), jnp.float32)
    return jnp.concatenate([As, z8, Ad, z8], axis=1)


def kernel(x, edge_index, W1, as1, ad1, b1, W2, as2, ad2, b2, W3, b3):
    n, kin = x.shape
    nc = n // CH

    loop = jnp.arange(n, dtype=edge_index.dtype)
    src_all = jnp.concatenate([edge_index[0], loop])
    dst_all = jnp.concatenate([edge_index[1], loop])
    order = jnp.argsort(dst_all)
    src_s = src_all[order].astype(jnp.int32)
    dst_s = dst_all[order].astype(jnp.int32)
    bnd = jnp.searchsorted(
        dst_s, jnp.arange(nc + 1, dtype=jnp.int32) * CH).astype(jnp.int32)
    bnd = jnp.pad(bnd, (0, 160 - (nc + 1)))
    src_p = jnp.concatenate([src_s, jnp.zeros((B,), jnp.int32)])
    dst_p = jnp.concatenate([dst_s, jnp.full((B,), n, jnp.int32)])

    xp = jnp.pad(x, ((0, 0), (0, 128 - kin)))
    W1p = jnp.pad(W1, ((0, 128 - kin), (0, 0)))
    Amat1 = _att_mat(as1, ad1)
    Amat2 = _att_mat(as2, ad2)
    W3p = jnp.pad(W3, ((0, 0), (0, 128 - W3.shape[1])))
    b3p = jnp.pad(b3, (0, 128 - b3.shape[0])).reshape(1, 128)

    T1, D1 = _mm_att(xp, W1p, Amat1, 400)
    g1 = _gat_sc(T1, src_p, dst_p, D1.reshape(-1), bnd, b1, n).reshape(n, F)
    T2, D2 = _mm_att(g1, W2, Amat2, 400)
    g2 = _gat_sc(T2, src_p, dst_p, D2.reshape(-1), bnd, b2, n).reshape(n, F)
    y = _final_mm(g2, W3p, b3p, 400)
    return y[:, : W3.shape[1]]


# speedup vs baseline: 5.3905x; 1.2525x over previous
import functools

import jax
import jax.numpy as jnp
from jax import lax
from jax.experimental import pallas as pl
from jax.experimental.pallas import tpu as pltpu
from jax.experimental.pallas import tpu_sc as plsc

HEADS = 8
C = 128
F = HEADS * C
DW = F + 128
CH = 80
NW = 32
B = 8


def _mm_att(xin, W, Amat, bn):
    n, k = xin.shape

    def body(x_ref, w_ref, a_ref, t_ref, d_ref):
        h = jnp.dot(x_ref[...], w_ref[...], preferred_element_type=jnp.float32)
        aa = jnp.dot(h, a_ref[...], preferred_element_type=jnp.float32)
        t_ref[:, :F] = h
        t_ref[:, F:F + 16] = aa[:, 0:16]
        d_ref[...] = aa[:, 16:32]

    return pl.pallas_call(
        body,
        grid=(n // bn,),
        in_specs=[
            pl.BlockSpec((bn, k), lambda i: (i, 0)),
            pl.BlockSpec((k, F), lambda i: (0, 0)),
            pl.BlockSpec((F, 32), lambda i: (0, 0)),
        ],
        out_specs=[
            pl.BlockSpec((bn, DW), lambda i: (i, 0)),
            pl.BlockSpec((bn, 16), lambda i: (i, 0)),
        ],
        out_shape=[
            jax.ShapeDtypeStruct((n, DW), jnp.float32),
            jax.ShapeDtypeStruct((n, 16), jnp.float32),
        ],
    )(xin, W, Amat)


def _final_mm(g, W3p, b3p, bn):
    n, k = g.shape

    def body(x_ref, w_ref, b_ref, o_ref):
        o_ref[...] = (
            jnp.dot(x_ref[...], w_ref[...], preferred_element_type=jnp.float32)
            + b_ref[...]
        )

    return pl.pallas_call(
        body,
        grid=(n // bn,),
        in_specs=[
            pl.BlockSpec((bn, k), lambda i: (i, 0)),
            pl.BlockSpec((k, 128), lambda i: (0, 0)),
            pl.BlockSpec((1, 128), lambda i: (0, 0)),
        ],
        out_specs=pl.BlockSpec((bn, 128), lambda i: (i, 0)),
        out_shape=jax.ShapeDtypeStruct((n, 128), jnp.float32),
    )(g, W3p, b3p)


def _gat_sc(T, srcP, dstP, adflat, bnd, bias, n):
    nc = n // CH
    ncw = (nc + NW - 1) // NW

    mesh = plsc.VectorSubcoreMesh(
        core_axis_name="c", subcore_axis_name="s", num_cores=2, num_subcores=16)

    @functools.partial(
        pl.kernel,
        out_type=jax.ShapeDtypeStruct((n * F,), jnp.float32),
        mesh=mesh,
        scratch_types=[
            pltpu.VMEM((CH * F,), jnp.float32),
            pltpu.VMEM((CH * 16,), jnp.float32),
            pltpu.VMEM((CH * 16,), jnp.float32),
            pltpu.VMEM((B, DW), jnp.float32),
            pltpu.VMEM((B, DW), jnp.float32),
            pltpu.VMEM((B,), jnp.int32),
            pltpu.VMEM((B,), jnp.int32),
            pltpu.VMEM((16,), jnp.int32),
            pltpu.VMEM((160,), jnp.int32),
            pltpu.VMEM((F,), jnp.float32),
            pltpu.SemaphoreType.DMA,
            pltpu.SemaphoreType.DMA,
        ],
    )
    def sc_kernel(t_h, src_h, dst_h, ad_h, bnd_h, b_h, g_h,
                  table, denomt, adstv, rows0, rows1, idxv0, idxv1, dstv,
                  bndv, biasv, sem0, sem1):
        cid = lax.axis_index("c")
        sid = lax.axis_index("s")
        wid = sid * 2 + cid
        pltpu.sync_copy(bnd_h, bndv)
        pltpu.sync_copy(b_h, biasv)

        @pl.loop(0, ncw)
        def _chunk_iter(ci):
            chunk = wid + ci * NW

            @pl.when(chunk < nc)
            def _():
                base = chunk * CH
                e01 = bndv[pl.ds(chunk, 16)]
                e0 = e01[0]
                e1 = e01[1]
                pltpu.sync_copy(ad_h.at[pl.ds(base * 16, CH * 16)], adstv)

                @pl.loop(0, CH * (F // 16))
                def _zero_table(i):
                    table[pl.ds(i * 16, 16)] = jnp.zeros((16,), jnp.float32)

                @pl.loop(0, CH)
                def _zero_denom(d):
                    denomt[pl.ds(d * 16, 16)] = jnp.zeros((16,), jnp.float32)

                e0a = (e0 // B) * B
                nb = (e1 - e0a + (B - 1)) // B

                pltpu.sync_copy(src_h.at[pl.ds(e0a, B)], idxv0)
                pltpu.async_copy(t_h.at[idxv0], rows0, sem0)
                pltpu.sync_copy(src_h.at[pl.ds(e0a + B, B)], idxv1)
                pltpu.async_copy(t_h.at[idxv1], rows1, sem1)

                @pl.loop(0, (nb + 1) // 2)
                def _bpair(bp):
                    for ph, idxv, rows, sem in (
                            (0, idxv0, rows0, sem0), (1, idxv1, rows1, sem1)):
                        bi = bp * 2 + ph

                        @pl.when(bi < nb)
                        def _():
                            k = e0a + bi * B
                            pltpu.make_async_copy(
                                t_h.at[idxv], rows, sem).wait()
                            pltpu.sync_copy(
                                dst_h.at[pl.ds(k, B)], dstv.at[pl.ds(0, B)])
                            dv = dstv[...]
                            for j in range(B):
                                local = dv[j] - base

                                @pl.when(jnp.logical_and(
                                    local >= 0, local < CH))
                                def _():
                                    tail = rows[j, pl.ds(F, 16)]
                                    adr = adstv[pl.ds(local * 16, 16)]
                                    alpha = tail + adr
                                    alpha = jnp.where(
                                        alpha >= 0.0, alpha, 0.2 * alpha)
                                    ex = jnp.exp(alpha)
                                    plsc.addupdate(
                                        denomt.at[pl.ds(local * 16, 16)], ex)
                                    toff = local * F
                                    for h in range(HEADS):
                                        bc = jnp.full((16,), ex[h],
                                                      jnp.float32)
                                        for c in range(C // 16):
                                            off = h * C + c * 16
                                            plsc.addupdate(
                                                table.at[pl.ds(toff + off, 16)],
                                                bc * rows[j, pl.ds(off, 16)])

                            @pl.when(bi + 2 < nb)
                            def _():
                                pltpu.sync_copy(
                                    src_h.at[pl.ds(k + 2 * B, B)], idxv)
                                pltpu.async_copy(t_h.at[idxv], rows, sem)

                @pl.loop(0, CH)
                def _finalize(d):
                    dn = denomt[pl.ds(d * 16, 16)]
                    rec = 1.0 / (dn + 1e-16)
                    for h in range(HEADS):
                        bc = jnp.full((16,), rec[h], jnp.float32)
                        for c in range(C // 16):
                            off = h * C + c * 16
                            v = (table[pl.ds(d * F + off, 16)] * bc
                                 + biasv[pl.ds(off, 16)])
                            v = jnp.where(v > 0.0, v, jnp.exp(v) - 1.0)
                            table[pl.ds(d * F + off, 16)] = v

                pltpu.sync_copy(table, g_h.at[pl.ds(base * F, CH * F)])

    return sc_kernel(T, srcP, dstP, adflat, bnd, bias)


def _att_mat(a_s, a_d):
    eye = jnp.eye(HEADS, dtype=jnp.float32)
    As = jnp.einsum("hc,hk->hck", a_s, eye).reshape(F, HEADS)
    Ad = jnp.einsum("hc,hk->hck", a_d, eye).reshape(F, HEADS)
    z8 = jnp.zeros((F, HEADS), jnp.float32)
    return jnp.concatenate([As, z8, Ad, z8], axis=1)


def kernel(x, edge_index, W1, as1, ad1, b1, W2, as2, ad2, b2, W3, b3):
    n, kin = x.shape
    nc = n // CH

    loop = jnp.arange(n, dtype=edge_index.dtype)
    src_all = jnp.concatenate([edge_index[0], loop])
    dst_all = jnp.concatenate([edge_index[1], loop])
    order = jnp.argsort(dst_all)
    src_s = src_all[order].astype(jnp.int32)
    dst_s = dst_all[order].astype(jnp.int32)
    bnd = jnp.searchsorted(
        dst_s, jnp.arange(nc + 1, dtype=jnp.int32) * CH).astype(jnp.int32)
    bnd = jnp.pad(bnd, (0, 160 - (nc + 1)))
    src_p = jnp.concatenate([src_s, jnp.zeros((B,), jnp.int32)])
    dst_p = jnp.concatenate([dst_s, jnp.full((B,), n, jnp.int32)])

    xp = jnp.pad(x, ((0, 0), (0, 128 - kin)))
    W1p = jnp.pad(W1, ((0, 128 - kin), (0, 0)))
    Amat1 = _att_mat(as1, ad1)
    Amat2 = _att_mat(as2, ad2)
    W3p = jnp.pad(W3, ((0, 0), (0, 128 - W3.shape[1])))
    b3p = jnp.pad(b3, (0, 128 - b3.shape[0])).reshape(1, 128)

    T1, D1 = _mm_att(xp, W1p, Amat1, 400)
    g1 = _gat_sc(T1, src_p, dst_p, D1.reshape(-1), bnd, b1, n).reshape(n, F)
    T2, D2 = _mm_att(g1, W2, Amat2, 400)
    g2 = _gat_sc(T2, src_p, dst_p, D2.reshape(-1), bnd, b2, n).reshape(n, F)
    y = _final_mm(g2, W3p, b3p, 400)
    return y[:, : W3.shape[1]]


# speedup vs baseline: 6.2195x; 1.1538x over previous
import functools

import jax
import jax.numpy as jnp
from jax import lax
from jax.experimental import pallas as pl
from jax.experimental.pallas import tpu as pltpu
from jax.experimental.pallas import tpu_sc as plsc

HEADS = 8
C = 128
F = HEADS * C
DW = F + 128
CH = 80
NW = 32
B = 8
NBLK = 32


def _mm_att(xin, W, Amat, bn):
    n, k = xin.shape

    def body(x_ref, w_ref, a_ref, t_ref, d_ref):
        h = jnp.dot(x_ref[...], w_ref[...], preferred_element_type=jnp.float32)
        aa = jnp.dot(h, a_ref[...], preferred_element_type=jnp.float32)
        t_ref[:, :F] = h
        t_ref[:, F:F + 16] = aa[:, 0:16]
        d_ref[...] = aa[:, 16:32]

    return pl.pallas_call(
        body,
        grid=(n // bn,),
        in_specs=[
            pl.BlockSpec((bn, k), lambda i: (i, 0)),
            pl.BlockSpec((k, F), lambda i: (0, 0)),
            pl.BlockSpec((F, 32), lambda i: (0, 0)),
        ],
        out_specs=[
            pl.BlockSpec((bn, DW), lambda i: (i, 0)),
            pl.BlockSpec((bn, 16), lambda i: (i, 0)),
        ],
        out_shape=[
            jax.ShapeDtypeStruct((n, DW), jnp.float32),
            jax.ShapeDtypeStruct((n, 16), jnp.float32),
        ],
    )(xin, W, Amat)


def _final_mm(g, W3p, b3p, bn):
    n, k = g.shape

    def body(x_ref, w_ref, b_ref, o_ref):
        o_ref[...] = (
            jnp.dot(x_ref[...], w_ref[...], preferred_element_type=jnp.float32)
            + b_ref[...]
        )

    return pl.pallas_call(
        body,
        grid=(n // bn,),
        in_specs=[
            pl.BlockSpec((bn, k), lambda i: (i, 0)),
            pl.BlockSpec((k, 128), lambda i: (0, 0)),
            pl.BlockSpec((1, 128), lambda i: (0, 0)),
        ],
        out_specs=pl.BlockSpec((bn, 128), lambda i: (i, 0)),
        out_shape=jax.ShapeDtypeStruct((n, 128), jnp.float32),
    )(g, W3p, b3p)


def _gat_sc(T, srcP, dstP, adflat, bnd, bias, n):
    nc = n // CH
    ncw = (nc + NW - 1) // NW

    mesh = plsc.VectorSubcoreMesh(
        core_axis_name="c", subcore_axis_name="s", num_cores=2, num_subcores=16)

    @functools.partial(
        pl.kernel,
        out_type=jax.ShapeDtypeStruct((n * F,), jnp.float32),
        mesh=mesh,
        scratch_types=[
            pltpu.VMEM((CH * F,), jnp.float32),
            pltpu.VMEM((CH * 16,), jnp.float32),
            pltpu.VMEM((CH * 16,), jnp.float32),
            pltpu.VMEM((B, DW), jnp.float32),
            pltpu.VMEM((B, DW), jnp.float32),
            pltpu.VMEM((NBLK * B,), jnp.int32),
            pltpu.VMEM((NBLK * B + 16,), jnp.int32),
            pltpu.VMEM((160,), jnp.int32),
            pltpu.VMEM((F,), jnp.float32),
            pltpu.SemaphoreType.DMA,
            pltpu.SemaphoreType.DMA,
        ],
    )
    def sc_kernel(t_h, src_h, dst_h, ad_h, bnd_h, b_h, g_h,
                  table, denomt, adstv, rows0, rows1, idxblk, dstblk,
                  bndv, biasv, sem0, sem1):
        cid = lax.axis_index("c")
        sid = lax.axis_index("s")
        wid = sid * 2 + cid
        pltpu.sync_copy(bnd_h, bndv)
        pltpu.sync_copy(b_h, biasv)

        @pl.loop(0, ncw)
        def _chunk_iter(ci):
            chunk = wid + ci * NW

            @pl.when(chunk < nc)
            def _():
                base = chunk * CH
                e01 = bndv[pl.ds(chunk, 16)]
                e0 = e01[0]
                e1 = e01[1]
                pltpu.sync_copy(ad_h.at[pl.ds(base * 16, CH * 16)], adstv)

                @pl.loop(0, CH * (F // 16))
                def _zero_table(i):
                    table[pl.ds(i * 16, 16)] = jnp.zeros((16,), jnp.float32)

                @pl.loop(0, CH)
                def _zero_denom(d):
                    denomt[pl.ds(d * 16, 16)] = jnp.zeros((16,), jnp.float32)

                e0a = (e0 // B) * B
                nb = (e1 - e0a + (B - 1)) // B

                @pl.loop(0, (nb + NBLK - 1) // NBLK)
                def _blk(blk):
                    kk = e0a + blk * (NBLK * B)
                    pltpu.sync_copy(src_h.at[pl.ds(kk, NBLK * B)], idxblk)
                    pltpu.sync_copy(
                        dst_h.at[pl.ds(kk, NBLK * B)],
                        dstblk.at[pl.ds(0, NBLK * B)])
                    nbb = jnp.minimum(nb - blk * NBLK, NBLK)

                    pltpu.async_copy(
                        t_h.at[idxblk.at[pl.ds(0, B)]], rows0, sem0)

                    @pl.when(nbb > 1)
                    def _():
                        pltpu.async_copy(
                            t_h.at[idxblk.at[pl.ds(B, B)]], rows1, sem1)

                    @pl.loop(0, (nbb + 1) // 2)
                    def _bpair(bp):
                        for ph, rows, sem in (
                                (0, rows0, sem0), (1, rows1, sem1)):
                            bi = bp * 2 + ph

                            @pl.when(bi < nbb)
                            def _():
                                off = bi * B
                                pltpu.make_async_copy(
                                    t_h.at[idxblk.at[pl.ds(off, B)]],
                                    rows, sem).wait()
                                dv = dstblk[pl.ds(off, 16)]
                                for j in range(B):
                                    local = dv[j] - base
                                    inb = jnp.logical_and(
                                        local >= 0, local < CH)
                                    msk = jnp.where(inb, 1.0, 0.0)
                                    lc = jnp.minimum(
                                        jnp.maximum(local, 0), CH - 1)
                                    tail = rows[j, pl.ds(F, 16)]
                                    adr = adstv[pl.ds(lc * 16, 16)]
                                    alpha = tail + adr
                                    alpha = jnp.where(
                                        alpha >= 0.0, alpha, 0.2 * alpha)
                                    ex = jnp.exp(alpha) * msk
                                    plsc.addupdate(
                                        denomt.at[pl.ds(lc * 16, 16)], ex)
                                    toff = lc * F
                                    for h in range(HEADS):
                                        bc = jnp.full((16,), ex[h],
                                                      jnp.float32)
                                        for c in range(C // 16):
                                            o2 = h * C + c * 16
                                            plsc.addupdate(
                                                table.at[pl.ds(toff + o2, 16)],
                                                bc * rows[j, pl.ds(o2, 16)])

                                @pl.when(bi + 2 < nbb)
                                def _():
                                    pltpu.async_copy(
                                        t_h.at[idxblk.at[
                                            pl.ds(off + 2 * B, B)]],
                                        rows, sem)

                @pl.loop(0, CH)
                def _finalize(d):
                    dn = denomt[pl.ds(d * 16, 16)]
                    rec = 1.0 / (dn + 1e-16)
                    for h in range(HEADS):
                        bc = jnp.full((16,), rec[h], jnp.float32)
                        for c in range(C // 16):
                            off = h * C + c * 16
                            v = (table[pl.ds(d * F + off, 16)] * bc
                                 + biasv[pl.ds(off, 16)])
                            v = jnp.where(v > 0.0, v, jnp.exp(v) - 1.0)
                            table[pl.ds(d * F + off, 16)] = v

                pltpu.sync_copy(table, g_h.at[pl.ds(base * F, CH * F)])

    return sc_kernel(T, srcP, dstP, adflat, bnd, bias)


def _att_mat(a_s, a_d):
    eye = jnp.eye(HEADS, dtype=jnp.float32)
    As = jnp.einsum("hc,hk->hck", a_s, eye).reshape(F, HEADS)
    Ad = jnp.einsum("hc,hk->hck", a_d, eye).reshape(F, HEADS)
    z8 = jnp.zeros((F, HEADS), jnp.float32)
    return jnp.concatenate([As, z8, Ad, z8], axis=1)


def kernel(x, edge_index, W1, as1, ad1, b1, W2, as2, ad2, b2, W3, b3):
    n, kin = x.shape
    nc = n // CH

    loop = jnp.arange(n, dtype=edge_index.dtype)
    src_all = jnp.concatenate([edge_index[0], loop])
    dst_all = jnp.concatenate([edge_index[1], loop])
    order = jnp.argsort(dst_all)
    src_s = src_all[order].astype(jnp.int32)
    dst_s = dst_all[order].astype(jnp.int32)
    bnd = jnp.searchsorted(
        dst_s, jnp.arange(nc + 1, dtype=jnp.int32) * CH).astype(jnp.int32)
    bnd = jnp.pad(bnd, (0, 160 - (nc + 1)))
    pad = NBLK * B + B
    src_p = jnp.concatenate([src_s, jnp.zeros((pad,), jnp.int32)])
    dst_p = jnp.concatenate([dst_s, jnp.full((pad,), n, jnp.int32)])

    xp = jnp.pad(x, ((0, 0), (0, 128 - kin)))
    W1p = jnp.pad(W1, ((0, 128 - kin), (0, 0)))
    Amat1 = _att_mat(as1, ad1)
    Amat2 = _att_mat(as2, ad2)
    W3p = jnp.pad(W3, ((0, 0), (0, 128 - W3.shape[1])))
    b3p = jnp.pad(b3, (0, 128 - b3.shape[0])).reshape(1, 128)

    T1, D1 = _mm_att(xp, W1p, Amat1, 400)
    g1 = _gat_sc(T1, src_p, dst_p, D1.reshape(-1), bnd, b1, n).reshape(n, F)
    T2, D2 = _mm_att(g1, W2, Amat2, 400)
    g2 = _gat_sc(T2, src_p, dst_p, D2.reshape(-1), bnd, b2, n).reshape(n, F)
    y = _final_mm(g2, W3p, b3p, 400)
    return y[:, : W3.shape[1]]
